# baseline (device time: 76571 ns/iter reference)
import os

import jax
import jax.numpy as jnp
from jax import lax
from jax.experimental import pallas as pl
from jax.experimental.pallas import tpu as pltpu

N_DEV = 16
SQ = 1024
D_MODEL = 1024
HQ_LOC = 8
DH = 128
BLK = 64
NGRP = 4
CH = SQ // N_DEV
SCALE = 0.08838834764831843

XOR_MASKS = (1, 3, 4, 8)
NPARTS = 4
PART_ROWS = SQ // NPARTS
RS_SIZES = (128, 64, 32, 16)
RS_REGIONS = (0, 128, 192, 224)
PART_STAGE = 240
ORDERS = tuple(
    tuple((p + r) % 4 for r in range(4)) for p in range(4)
)

DO_COMM = os.environ.get("KERNEL_NO_COMM") != "1"


def kernel(x, Wq, K_ext, V_ext, Wo):
    my = lax.axis_index("i")
    x2 = x[0]
    k_loc = jnp.transpose(
        lax.dynamic_slice_in_dim(K_ext[0], my * HQ_LOC, HQ_LOC, axis=1),
        (1, 0, 2),
    )
    v_loc = jnp.transpose(
        lax.dynamic_slice_in_dim(V_ext[0], my * HQ_LOC, HQ_LOC, axis=1),
        (1, 0, 2),
    )

    def body(x_ref, wq_ref, k_ref, v_ref, wo_ref, out_ref,
             comm_ref, outb_ref, send_sems, recv_sems):
        me = lax.axis_index("i")
        partners = [jnp.bitwise_xor(me, m) for m in XOR_MASKS]
        w = jnp.mod(me, 4)
        bits = [
            jnp.logical_or(w == 1, w == 2).astype(jnp.int32),
            (w >= 2).astype(jnp.int32),
            jnp.mod(me // 4, 2),
            me // 8,
        ]

        if DO_COMM:
            barrier_sem = pltpu.get_barrier_semaphore()
            for nbr in partners:
                pl.semaphore_signal(
                    barrier_sem, inc=1,
                    device_id=(nbr,), device_id_type=pl.DeviceIdType.MESH,
                )
            pl.semaphore_wait(barrier_sem, 4)

        cur_off = [jnp.int32(PART_ROWS * p) for p in range(NPARTS)]
        blk_off = [None] * NPARTS
        inflight = {}

        def rs_start(p, r):
            s = RS_SIZES[r]
            b = bits[ORDERS[p][r]]
            send_off = cur_off[p] + (1 - b) * s
            cur_off[p] = cur_off[p] + b * s
            r0 = PART_STAGE * p + RS_REGIONS[r]
            rdma = pltpu.make_async_remote_copy(
                src_ref=outb_ref.at[pl.ds(send_off, s), :],
                dst_ref=comm_ref.at[pl.ds(r0, s), :],
                send_sem=send_sems.at[NPARTS * r + p],
                recv_sem=recv_sems.at[NPARTS * r + p],
                device_id=(partners[ORDERS[p][r]],),
                device_id_type=pl.DeviceIdType.MESH,
            )
            rdma.start()
            inflight[("rs", p, r)] = rdma

        def rs_fin(p, r):
            inflight.pop(("rs", p, r)).wait()
            s = RS_SIZES[r]
            r0 = PART_STAGE * p + RS_REGIONS[r]
            sl = pl.ds(cur_off[p], s)
            acc = out_ref[sl, :] + comm_ref[r0:r0 + s, :].astype(jnp.float32)
            out_ref[sl, :] = acc
            outb_ref[sl, :] = acc.astype(jnp.bfloat16)
            if r == 3:
                blk_off[p] = cur_off[p]

        def ag_start(p, j):
            r = 3 - j
            s = RS_SIZES[r]
            b = bits[ORDERS[p][r]]
            rdma = pltpu.make_async_remote_copy(
                src_ref=outb_ref.at[pl.ds(blk_off[p], s), :],
                dst_ref=outb_ref.at[pl.ds(blk_off[p], s), :],
                send_sem=send_sems.at[NPARTS * (4 + j) + p],
                recv_sem=recv_sems.at[NPARTS * (4 + j) + p],
                device_id=(partners[ORDERS[p][r]],),
                device_id_type=pl.DeviceIdType.MESH,
            )
            rdma.start()
            inflight[("ag", p, j)] = rdma
            blk_off[p] = blk_off[p] - b * s

        def ag_fin(p, j):
            inflight.pop(("ag", p, j)).wait()

        ACTIONS = {"rs_s": rs_start, "rs_f": rs_fin,
                   "ag_s": ag_start, "ag_f": ag_fin}
        AFTER_GROUP = (
            (("rs_s", 0, 0),),
            (("rs_s", 1, 0),),
            (("rs_s", 2, 0),),
            (("rs_s", 3, 0),),
        )
        DRAIN = tuple(
            (act, p, 0) for act in ("rs_f",) for p in range(NPARTS)
        ) + tuple(
            (act, p, r)
            for r in range(1, 4)
            for act in ("rs_s", "rs_f")
            for p in range(NPARTS)
        ) + tuple(
            (act, p, j)
            for j in range(4)
            for act in ("ag_s", "ag_f")
            for p in range(NPARTS)
        )

        for g in range(NGRP):
            rows = [BLK * (g + NGRP * k) for k in range(4)]
            x_r = jnp.concatenate(
                [x_ref[pl.ds(s, BLK), :] for s in rows], axis=0
            )
            q_r = jnp.dot(x_r, wq_ref[:, :],
                          preferred_element_type=jnp.float32)
            ctx = []
            for h in range(HQ_LOC):
                k_h = jnp.concatenate(
                    [k_ref[h, pl.ds(s, BLK), :] for s in rows], axis=0
                )
                v_h = jnp.concatenate(
                    [v_ref[h, pl.ds(s, BLK), :] for s in rows], axis=0
                )
                s_rh = jnp.dot(q_r[:, h * DH:(h + 1) * DH], k_h.T,
                               preferred_element_type=jnp.float32) * SCALE
                m = jnp.max(s_rh, axis=1, keepdims=True)
                wgt = jnp.exp(s_rh - m)
                wgt = wgt / jnp.sum(wgt, axis=1, keepdims=True)
                ctx.append(jnp.dot(wgt, v_h,
                                   preferred_element_type=jnp.float32))
            ctx_r = jnp.concatenate(ctx, axis=1)
            p_r = jnp.dot(ctx_r, wo_ref[:, :],
                          preferred_element_type=jnp.float32)
            out_ref[pl.ds(PART_ROWS * g, PART_ROWS), :] = p_r
            outb_ref[pl.ds(PART_ROWS * g, PART_ROWS), :] = p_r.astype(
                jnp.bfloat16
            )
            if DO_COMM:
                for act, p, r in AFTER_GROUP[g]:
                    ACTIONS[act](p, r)

        if not DO_COMM:
            return
        for act, p, r in DRAIN:
            ACTIONS[act](p, r)

        for g in range(NGRP):
            for k in range(4):
                out_ref[pl.ds(BLK * (g + NGRP * k), BLK), :] = outb_ref[
                    pl.ds(PART_ROWS * g + BLK * k, BLK), :
                ].astype(jnp.float32)

    out = pl.pallas_call(
        body,
        out_shape=jax.ShapeDtypeStruct((SQ, D_MODEL), jnp.float32),
        in_specs=[pl.BlockSpec(memory_space=pltpu.VMEM)] * 5,
        out_specs=pl.BlockSpec(memory_space=pltpu.VMEM),
        scratch_shapes=[
            pltpu.VMEM((960, D_MODEL), jnp.bfloat16),
            pltpu.VMEM((SQ, D_MODEL), jnp.bfloat16),
            pltpu.SemaphoreType.DMA((32,)),
            pltpu.SemaphoreType.DMA((32,)),
        ],
        compiler_params=pltpu.CompilerParams(
            collective_id=0 if DO_COMM else None,
            vmem_limit_bytes=100 * 1024 * 1024,
        ),
    )(x2, Wq, k_loc, v_loc, Wo)
    return out[None, :, :]


# device time: 67344 ns/iter; 1.1370x vs baseline; 1.1370x over previous
import os

import jax
import jax.numpy as jnp
from jax import lax
from jax.experimental import pallas as pl
from jax.experimental.pallas import tpu as pltpu

N_DEV = 16
SQ = 1024
D_MODEL = 1024
HQ_LOC = 8
DH = 128
BLK = 64
NGRP = 4
CH = SQ // N_DEV
SCALE = 0.08838834764831843

XOR_MASKS = (1, 3, 4, 8)
NPARTS = 4
PART_ROWS = SQ // NPARTS
RS_SIZES = (128, 64, 32, 16)
RS_REGIONS = (0, 128, 192, 224)
PART_STAGE = 240
ORDERS = tuple(
    tuple((p + r) % 4 for r in range(4)) for p in range(4)
)

DO_COMM = os.environ.get("KERNEL_NO_COMM") != "1"
DO_COMPUTE = os.environ.get("KERNEL_NO_COMPUTE") != "1"


def kernel(x, Wq, K_ext, V_ext, Wo):
    my = lax.axis_index("i")
    x2 = x[0]
    k_loc = jnp.transpose(
        lax.dynamic_slice_in_dim(K_ext[0], my * HQ_LOC, HQ_LOC, axis=1),
        (1, 0, 2),
    )
    v_loc = jnp.transpose(
        lax.dynamic_slice_in_dim(V_ext[0], my * HQ_LOC, HQ_LOC, axis=1),
        (1, 0, 2),
    )

    def body(x_ref, wq_ref, k_ref, v_ref, wo_ref, out_ref,
             comm_ref, outb_ref, send_sems, recv_sems):
        me = lax.axis_index("i")
        partners = [jnp.bitwise_xor(me, m) for m in XOR_MASKS]
        w = jnp.mod(me, 4)
        bits = [
            jnp.logical_or(w == 1, w == 2).astype(jnp.int32),
            (w >= 2).astype(jnp.int32),
            jnp.mod(me // 4, 2),
            me // 8,
        ]

        if DO_COMM:
            barrier_sem = pltpu.get_barrier_semaphore()
            for nbr in partners:
                pl.semaphore_signal(
                    barrier_sem, inc=1,
                    device_id=(nbr,), device_id_type=pl.DeviceIdType.MESH,
                )
            pl.semaphore_wait(barrier_sem, 4)

        cur_off = [jnp.int32(PART_ROWS * p) for p in range(NPARTS)]
        blk_off = [None] * NPARTS
        inflight = {}

        def rs_start(p, r):
            s = RS_SIZES[r]
            b = bits[ORDERS[p][r]]
            send_off = cur_off[p] + (1 - b) * s
            cur_off[p] = cur_off[p] + b * s
            r0 = PART_STAGE * p + RS_REGIONS[r]
            rdma = pltpu.make_async_remote_copy(
                src_ref=outb_ref.at[pl.ds(send_off, s), :],
                dst_ref=comm_ref.at[pl.ds(r0, s), :],
                send_sem=send_sems.at[NPARTS * r + p],
                recv_sem=recv_sems.at[NPARTS * r + p],
                device_id=(partners[ORDERS[p][r]],),
                device_id_type=pl.DeviceIdType.MESH,
            )
            rdma.start()
            inflight[("rs", p, r)] = rdma

        def rs_fin(p, r):
            inflight.pop(("rs", p, r)).wait()
            s = RS_SIZES[r]
            r0 = PART_STAGE * p + RS_REGIONS[r]
            sl = pl.ds(cur_off[p], s)
            acc = out_ref[sl, :] + comm_ref[r0:r0 + s, :].astype(jnp.float32)
            out_ref[sl, :] = acc
            outb_ref[sl, :] = acc.astype(jnp.bfloat16)
            if r == 3:
                blk_off[p] = cur_off[p]

        def ag_start(p, j):
            r = 3 - j
            s = RS_SIZES[r]
            b = bits[ORDERS[p][r]]
            rdma = pltpu.make_async_remote_copy(
                src_ref=outb_ref.at[pl.ds(blk_off[p], s), :],
                dst_ref=outb_ref.at[pl.ds(blk_off[p], s), :],
                send_sem=send_sems.at[NPARTS * (4 + j) + p],
                recv_sem=recv_sems.at[NPARTS * (4 + j) + p],
                device_id=(partners[ORDERS[p][r]],),
                device_id_type=pl.DeviceIdType.MESH,
            )
            rdma.start()
            inflight[("ag", p, j)] = rdma
            blk_off[p] = blk_off[p] - b * s

        def ag_fin(p, j):
            inflight.pop(("ag", p, j)).wait()

        ACTIONS = {"rs_s": rs_start, "rs_f": rs_fin,
                   "ag_s": ag_start, "ag_f": ag_fin}
        AFTER_GROUP = (
            (("rs_s", 0, 0),),
            (("rs_s", 1, 0),),
            (("rs_s", 2, 0),),
            (("rs_s", 3, 0),),
        )
        DRAIN = tuple(
            (act, p, 0) for act in ("rs_f",) for p in range(NPARTS)
        ) + tuple(
            (act, p, r)
            for r in range(1, 4)
            for act in ("rs_s", "rs_f")
            for p in range(NPARTS)
        ) + tuple(
            (act, p, j)
            for j in range(4)
            for act in ("ag_s", "ag_f")
            for p in range(NPARTS)
        )

        for g in range(NGRP if DO_COMPUTE else 0):
            rows = [BLK * (g + NGRP * k) for k in range(4)]
            x_r = jnp.concatenate(
                [x_ref[pl.ds(s, BLK), :] for s in rows], axis=0
            )
            q_r = jnp.dot(x_r, wq_ref[:, :],
                          preferred_element_type=jnp.float32)
            ctx = []
            for h in range(HQ_LOC):
                k_h = jnp.concatenate(
                    [k_ref[h, pl.ds(s, BLK), :] for s in rows], axis=0
                )
                v_h = jnp.concatenate(
                    [v_ref[h, pl.ds(s, BLK), :] for s in rows], axis=0
                )
                s_rh = jnp.dot(q_r[:, h * DH:(h + 1) * DH], k_h.T,
                               preferred_element_type=jnp.float32) * SCALE
                m = jnp.max(s_rh, axis=1, keepdims=True)
                wgt = jnp.exp(s_rh - m)
                wgt = wgt / jnp.sum(wgt, axis=1, keepdims=True)
                ctx.append(jnp.dot(wgt, v_h,
                                   preferred_element_type=jnp.float32))
            ctx_r = jnp.concatenate(ctx, axis=1)
            p_r = jnp.dot(ctx_r, wo_ref[:, :],
                          preferred_element_type=jnp.float32)
            out_ref[pl.ds(PART_ROWS * g, PART_ROWS), :] = p_r
            outb_ref[pl.ds(PART_ROWS * g, PART_ROWS), :] = p_r.astype(
                jnp.bfloat16
            )
            if DO_COMM:
                for act, p, r in AFTER_GROUP[g]:
                    ACTIONS[act](p, r)

        if not DO_COMM:
            return
        if not DO_COMPUTE:
            for steps in AFTER_GROUP:
                for act, p, r in steps:
                    ACTIONS[act](p, r)
        for act, p, r in DRAIN:
            ACTIONS[act](p, r)

        for g in range(NGRP):
            for k in range(4):
                out_ref[pl.ds(BLK * (g + NGRP * k), BLK), :] = outb_ref[
                    pl.ds(PART_ROWS * g + BLK * k, BLK), :
                ].astype(jnp.float32)

    out = pl.pallas_call(
        body,
        out_shape=jax.ShapeDtypeStruct((SQ, D_MODEL), jnp.float32),
        in_specs=[pl.BlockSpec(memory_space=pltpu.VMEM)] * 5,
        out_specs=pl.BlockSpec(memory_space=pltpu.VMEM),
        scratch_shapes=[
            pltpu.VMEM((960, D_MODEL), jnp.bfloat16),
            pltpu.VMEM((SQ, D_MODEL), jnp.bfloat16),
            pltpu.SemaphoreType.DMA((32,)),
            pltpu.SemaphoreType.DMA((32,)),
        ],
        compiler_params=pltpu.CompilerParams(
            collective_id=0 if DO_COMM else None,
            vmem_limit_bytes=100 * 1024 * 1024,
        ),
    )(x2, Wq, k_loc, v_loc, Wo)
    return out[None, :, :]
